# baseline (speedup 1.0000x reference)
import functools

import jax
import jax.numpy as jnp
from jax import lax
from jax.experimental import pallas as pl
from jax.experimental.pallas import tpu as pltpu
from jax.experimental.pallas import tpu_sc as plsc

SIGMA = 9.0
N = 200000
NW = 32
CHUNK = 6256
LAST = N - (NW - 1) * CHUNK
ITERS = CHUNK // 16


def _sc_partials(inp2d, tgt2d):
    mesh = plsc.VectorSubcoreMesh(core_axis_name="c", subcore_axis_name="s")

    @functools.partial(
        pl.kernel,
        mesh=mesh,
        compiler_params=pltpu.CompilerParams(needs_layout_passes=False),
        out_type=[
            jax.ShapeDtypeStruct((NW, 16), jnp.float32),
            jax.ShapeDtypeStruct((NW, 16), jnp.float32),
        ],
        scratch_types=[
            pltpu.VMEM((CHUNK * 2,), jnp.float32),
            pltpu.VMEM((CHUNK * 3,), jnp.float32),
            pltpu.VMEM((16,), jnp.float32),
            pltpu.VMEM((16,), jnp.float32),
        ],
    )
    def body(inp_hbm, tgt_hbm, loss_out, cnt_out, inp_v, tgt_v, acc_v, cntacc_v):
        wid = lax.axis_index("s") * 2 + lax.axis_index("c")
        base = wid * CHUNK

        @pl.when(wid < NW - 1)
        def _():
            pltpu.sync_copy(inp_hbm.at[pl.ds(base * 2, CHUNK * 2)], inp_v)
            pltpu.sync_copy(tgt_hbm.at[pl.ds(base * 3, CHUNK * 3)], tgt_v)

        @pl.when(wid == NW - 1)
        def _():
            pltpu.sync_copy(inp_hbm.at[pl.ds(base * 2, LAST * 2)],
                            inp_v.at[pl.ds(0, LAST * 2)])
            pltpu.sync_copy(tgt_hbm.at[pl.ds(base * 3, LAST * 3)],
                            tgt_v.at[pl.ds(0, LAST * 3)])

        lane = lax.iota(jnp.int32, 16)
        limit = jnp.where(wid == NW - 1, LAST, CHUNK)

        def step(i, carry):
            acc, cnt = carry
            rows = lane + i * 16
            valid = rows < limit
            r3 = rows * 3
            r2 = rows * 2
            cls = plsc.load_gather(tgt_v, [r3])
            tx = plsc.load_gather(tgt_v, [r3 + 1])
            ty = plsc.load_gather(tgt_v, [r3 + 2])
            px = plsc.load_gather(inp_v, [r2])
            py = plsc.load_gather(inp_v, [r2 + 1])
            dx = jnp.abs(tx - px)
            dy = jnp.abs(ty - py)
            fx = jnp.where(dx < 1.0 / SIGMA, 0.5 * SIGMA * dx * dx,
                           dx - 0.5 / SIGMA)
            fy = jnp.where(dy < 1.0 / SIGMA, 0.5 * SIGMA * dy * dy,
                           dy - 0.5 / SIGMA)
            m = jnp.where(jnp.logical_and(cls == 1.0, valid), 1.0, 0.0)
            return acc + m * (fx + fy), cnt + m

        zero = jnp.zeros((16,), jnp.float32)
        acc, cnt = lax.fori_loop(0, ITERS, step, (zero, zero))
        acc_v[...] = acc
        cntacc_v[...] = cnt
        pltpu.sync_copy(acc_v, loss_out.at[wid])
        pltpu.sync_copy(cntacc_v, cnt_out.at[wid])

    return body(inp2d, tgt2d)


def _finish(loss_p, cnt_p):
    def body(loss_ref, cnt_ref, o_ref):
        total = jnp.sum(loss_ref[...])
        count = jnp.sum(cnt_ref[...])
        o_ref[0, 0] = jnp.where(count > 0.0,
                                total / jnp.maximum(count, 1.0),
                                jnp.float32(0.0))

    return pl.pallas_call(
        body,
        out_shape=jax.ShapeDtypeStruct((1, 1), jnp.float32),
        out_specs=pl.BlockSpec(memory_space=pltpu.SMEM),
    )(loss_p, cnt_p)


def kernel(input, target):
    inp2d = input.reshape(N * 2)
    tgt2d = target.reshape(N * 3)
    loss_p, cnt_p = _sc_partials(inp2d, tgt2d)
    return _finish(loss_p, cnt_p).reshape(())


# speedup vs baseline: 10.5396x; 10.5396x over previous
import functools

import jax
import jax.numpy as jnp
from jax import lax
from jax.experimental import pallas as pl
from jax.experimental.pallas import tpu as pltpu
from jax.experimental.pallas import tpu_sc as plsc

SIGMA = 9.0
N = 200000
NW = 32
CHUNK = 6272
LAST = N - (NW - 1) * CHUNK
LAST_PAD = 5632
ITERS = CHUNK // 16


def _sc_partials(xT, tT):
    mesh = plsc.VectorSubcoreMesh(core_axis_name="c", subcore_axis_name="s")

    @functools.partial(
        pl.kernel,
        mesh=mesh,
        compiler_params=pltpu.CompilerParams(needs_layout_passes=False,
                                             disable_bounds_checks=True),
        out_type=[
            jax.ShapeDtypeStruct((NW, 16), jnp.float32),
            jax.ShapeDtypeStruct((NW, 16), jnp.float32),
        ],
        scratch_types=[
            pltpu.VMEM((2, CHUNK), jnp.float32),
            pltpu.VMEM((CHUNK,), jnp.float32),
            pltpu.VMEM((CHUNK,), jnp.float32),
            pltpu.VMEM((CHUNK,), jnp.float32),
            pltpu.VMEM((16,), jnp.float32),
            pltpu.VMEM((16,), jnp.float32),
            pltpu.SemaphoreType.DMA,
        ],
    )
    def body(x_hbm, t_hbm, loss_out, cnt_out,
             xyv, clsv, txv, tyv, acc_v, cntacc_v, sem):
        wid = lax.axis_index("s") * 2 + lax.axis_index("c")
        base = wid * CHUNK

        def stage(n):
            cps = [
                pltpu.async_copy(x_hbm.at[0, :, pl.ds(base, n)],
                                 xyv.at[:, pl.ds(0, n)], sem),
                pltpu.async_copy(t_hbm.at[0, 0, pl.ds(base, n)],
                                 clsv.at[pl.ds(0, n)], sem),
                pltpu.async_copy(t_hbm.at[1, 0, pl.ds(base, n)],
                                 txv.at[pl.ds(0, n)], sem),
                pltpu.async_copy(t_hbm.at[2, 0, pl.ds(base, n)],
                                 tyv.at[pl.ds(0, n)], sem),
            ]
            for c in cps:
                c.wait()

        @pl.when(wid < NW - 1)
        def _():
            stage(CHUNK)

        @pl.when(wid == NW - 1)
        def _():
            stage(LAST_PAD)

        lane = lax.iota(jnp.int32, 16)
        limit = jnp.where(wid == NW - 1, LAST, CHUNK)

        def step(i, carry):
            acc, cnt = carry
            off = i * 16
            valid = (lane + off) < limit
            cls = clsv[pl.ds(off, 16)]
            tx = txv[pl.ds(off, 16)]
            ty = tyv[pl.ds(off, 16)]
            px = xyv[0, pl.ds(off, 16)]
            py = xyv[1, pl.ds(off, 16)]
            dx = jnp.abs(tx - px)
            dy = jnp.abs(ty - py)
            fx = jnp.where(dx < 1.0 / SIGMA, 0.5 * SIGMA * dx * dx,
                           dx - 0.5 / SIGMA)
            fy = jnp.where(dy < 1.0 / SIGMA, 0.5 * SIGMA * dy * dy,
                           dy - 0.5 / SIGMA)
            m = jnp.where(jnp.logical_and(cls == 1.0, valid), 1.0, 0.0)
            return acc + m * (fx + fy), cnt + m

        zero = jnp.zeros((16,), jnp.float32)
        acc, cnt = lax.fori_loop(0, ITERS, step, (zero, zero))
        acc_v[...] = acc
        cntacc_v[...] = cnt
        pltpu.sync_copy(acc_v, loss_out.at[wid])
        pltpu.sync_copy(cntacc_v, cnt_out.at[wid])

    return body(xT, tT)


def _finish(loss_p, cnt_p):
    def body(loss_ref, cnt_ref, o_ref):
        total = jnp.sum(loss_ref[...])
        count = jnp.sum(cnt_ref[...])
        o_ref[0, 0] = jnp.where(count > 0.0,
                                total / jnp.maximum(count, 1.0),
                                jnp.float32(0.0))

    return pl.pallas_call(
        body,
        out_shape=jax.ShapeDtypeStruct((1, 1), jnp.float32),
        out_specs=pl.BlockSpec(memory_space=pltpu.SMEM),
    )(loss_p, cnt_p)


def kernel(input, target):
    xT = jnp.transpose(input, (0, 2, 1))
    tT = jnp.transpose(target, (2, 0, 1))
    loss_p, cnt_p = _sc_partials(xT, tT)
    return _finish(loss_p, cnt_p).reshape(())
